# baseline (speedup 1.0000x reference)
import functools

import jax
import jax.numpy as jnp
from jax import lax
from jax.experimental import pallas as pl
from jax.experimental.pallas import tpu as pltpu
from jax.experimental.pallas import tpu_sc as plsc

_E = 8
_D = 768
_BT = 4096
_NSC = 8192
_C = 32
_NW = 32
_NG = _C // 16
_LANES = 16


def _tc_kernel(x_ref, wt_ref, gates_ref, idx_ref, usage_ref, acc_ref, *,
               nblocks):
    i = pl.program_id(0)
    logits = jnp.dot(x_ref[...], wt_ref[...],
                     preferred_element_type=jnp.float32)
    lt = logits.T
    bt = lt.shape[1]
    e = lax.broadcasted_iota(jnp.int32, (_E, bt), 0)

    m1 = jnp.max(lt, axis=0, keepdims=True)
    i1 = jnp.min(jnp.where(lt == m1, e, _E), axis=0, keepdims=True)
    masked = jnp.where(e == i1, -jnp.inf, lt)
    m2 = jnp.max(masked, axis=0, keepdims=True)
    i2 = jnp.min(jnp.where(masked == m2, e, _E), axis=0, keepdims=True)

    ed = jnp.exp(m2 - m1)
    g2 = ed / (1.0 + ed)
    g1 = 1.0 - g2

    gt = jnp.where(e == i1, g1, jnp.where(e == i2, g2, jnp.float32(0.0)))
    gates_ref[...] = gt.T
    idx_ref[...] = jnp.concatenate([i1, i2], axis=0).T

    @pl.when(i == 0)
    def _init():
        acc_ref[...] = jnp.zeros_like(acc_ref)

    acc_ref[...] += jnp.sum(gt, axis=1, keepdims=True)

    @pl.when(i == nblocks - 1)
    def _fin():
        usage_ref[...] = acc_ref[...]


def _sc_kernel(x_hbm, w_hbm, gates_hbm, idx_hbm, usage_hbm,
               w_v, x_v, g_v, i_v, u_v, *, per_w):
    nc = 2
    wid = lax.axis_index("s") * nc + lax.axis_index("c")
    base = wid * per_w
    pltpu.sync_copy(w_hbm, w_v)
    lane = lax.broadcasted_iota(jnp.int32, (_LANES,), 0)
    rows = [lane + _LANES * g for g in range(_NG)]
    nchunks = per_w // _C

    def chunk_body(ci, usage):
        tok0 = base + ci * _C
        pltpu.sync_copy(x_hbm.at[pl.ds(tok0, _C), :],
                        x_v.at[:, pl.ds(0, _D)])

        def _round_bf16(v):
            t = plsc.bitcast(v, jnp.int32)
            lsb = jnp.bitwise_and(jnp.right_shift(t, 16), 1)
            t = jnp.bitwise_and(t + 0x7FFF + lsb, jnp.int32(-65536))
            return plsc.bitcast(t, jnp.float32)

        def d_body(dc, accs):
            d0 = dc * _LANES
            wv = [_round_bf16(w_v[pl.ds(ei * _D + d0, _LANES)])
                  for ei in range(_E)]
            accs = list(accs)
            for j in range(_LANES):
                col = jnp.full((_LANES,), 0, jnp.int32) + (d0 + j)
                xg = [_round_bf16(plsc.load_gather(x_v, (rows[g], col)))
                      for g in range(_NG)]
                for ei in range(_E):
                    ws = wv[ei][j]
                    for g in range(_NG):
                        k = ei * _NG + g
                        accs[k] = accs[k] + xg[g] * ws
            return accs

        zero = jnp.zeros((_LANES,), jnp.float32)
        accs = lax.fori_loop(0, _D // _LANES, d_body, [zero] * (_E * _NG))

        new_usage = list(usage)
        for g in range(_NG):
            a = [accs[ei * _NG + g] for ei in range(_E)]
            m1 = a[0]
            for ei in range(1, _E):
                m1 = jnp.maximum(m1, a[ei])
            i1 = jnp.full((_LANES,), _E, jnp.int32)
            for ei in range(_E - 1, -1, -1):
                i1 = jnp.where(a[ei] == m1, ei, i1)
            neg = jnp.full((_LANES,), -jnp.inf, jnp.float32)
            m2 = neg
            for ei in range(_E):
                m2 = jnp.maximum(m2, jnp.where(i1 == ei, neg, a[ei]))
            i2 = jnp.full((_LANES,), _E, jnp.int32)
            for ei in range(_E - 1, -1, -1):
                hit = jnp.logical_and(a[ei] == m2, i1 != ei)
                i2 = jnp.where(hit, ei, i2)
            ed = jnp.exp(m2 - m1)
            g2v = ed / (1.0 + ed)
            g1v = 1.0 - g2v
            rg8 = rows[g] * _E
            for ei in range(_E):
                gate = jnp.where(i1 == ei, g1v,
                                 jnp.where(i2 == ei, g2v, jnp.float32(0.0)))
                plsc.store_scatter(g_v, (rg8 + ei,), gate)
                new_usage[ei] = new_usage[ei] + gate
            rg2 = rows[g] * 2
            plsc.store_scatter(i_v, (rg2,), i1)
            plsc.store_scatter(i_v, (rg2 + 1,), i2)
        pltpu.sync_copy(g_v, gates_hbm.at[pl.ds(tok0 * _E, _C * _E)])
        pltpu.sync_copy(i_v, idx_hbm.at[pl.ds(tok0 * 2, _C * 2)])
        return new_usage

    zero = jnp.zeros((_LANES,), jnp.float32)
    usage = lax.fori_loop(0, nchunks, chunk_body, [zero] * _E)
    for ei in range(_E):
        u_v[pl.ds(ei * _LANES, _LANES)] = usage[ei]
    pltpu.sync_copy(u_v, usage_hbm.at[pl.ds(wid * _E * _LANES, _E * _LANES)])


def _loss_kernel(ut_ref, us_ref, loss_ref, *, ntokens):
    u = ut_ref[...] + jnp.sum(us_ref[...], axis=(0, 2)).reshape(_E, 1)
    usage = u / jnp.float32(ntokens)
    log_usage = jnp.maximum(jnp.log(usage), -1e9)
    un = jnp.float32(1.0 / _E)
    loss_ref[...] = jnp.sum(un * (jnp.log(un) - log_usage)).reshape(1, 1)


def kernel(input_tensor, W):
    B, S, D = input_tensor.shape
    n = B * S
    x = input_tensor.reshape(n, D)
    wt = W.T
    ntc = n - _NSC
    nblocks = ntc // _BT
    per_w = _NSC // _NW

    tc_gates, tc_idx, tc_usage = pl.pallas_call(
        functools.partial(_tc_kernel, nblocks=nblocks),
        grid=(nblocks,),
        in_specs=[
            pl.BlockSpec((_BT, D), lambda i: (i, 0)),
            pl.BlockSpec((D, _E), lambda i: (0, 0)),
        ],
        out_specs=[
            pl.BlockSpec((_BT, _E), lambda i: (i, 0)),
            pl.BlockSpec((_BT, 2), lambda i: (i, 0)),
            pl.BlockSpec((_E, 1), lambda i: (0, 0)),
        ],
        out_shape=[
            jax.ShapeDtypeStruct((ntc, _E), jnp.float32),
            jax.ShapeDtypeStruct((ntc, 2), jnp.int32),
            jax.ShapeDtypeStruct((_E, 1), jnp.float32),
        ],
        scratch_shapes=[pltpu.VMEM((_E, 1), jnp.float32)],
    )(x[:ntc], wt)

    sc_fn = pl.kernel(
        functools.partial(_sc_kernel, per_w=per_w),
        out_type=[
            jax.ShapeDtypeStruct((_NSC * _E,), jnp.float32),
            jax.ShapeDtypeStruct((_NSC * 2,), jnp.int32),
            jax.ShapeDtypeStruct((_NW * _E * _LANES,), jnp.float32),
        ],
        mesh=plsc.VectorSubcoreMesh(core_axis_name="c", subcore_axis_name="s",
                                    num_cores=2, num_subcores=16),
        compiler_params=pltpu.CompilerParams(needs_layout_passes=False),
        scratch_types=[
            pltpu.VMEM((_E * _D,), jnp.float32),
            pltpu.VMEM((_C, _D + 1), jnp.float32),
            pltpu.VMEM((_C * _E,), jnp.float32),
            pltpu.VMEM((_C * 2,), jnp.int32),
            pltpu.VMEM((_E * _LANES,), jnp.float32),
        ],
    )
    sc_gates, sc_idx, sc_usage = sc_fn(x[ntc:], W.reshape(_E * _D))
    sc_gates = sc_gates.reshape(_NSC, _E)
    sc_idx = sc_idx.reshape(_NSC, 2)
    sc_usage = sc_usage.reshape(_NW, _E, _LANES)

    loss = pl.pallas_call(
        functools.partial(_loss_kernel, ntokens=n),
        in_specs=[
            pl.BlockSpec(memory_space=pltpu.VMEM),
            pl.BlockSpec(memory_space=pltpu.VMEM),
        ],
        out_specs=pl.BlockSpec(memory_space=pltpu.VMEM),
        out_shape=jax.ShapeDtypeStruct((1, 1), jnp.float32),
    )(tc_usage, sc_usage)

    gates = jnp.concatenate([tc_gates, sc_gates], axis=0)
    idx = jnp.concatenate([tc_idx, sc_idx], axis=0)
    return (gates.reshape(B, S, _E), idx.reshape(B, S, 2), loss.reshape(()))


# speedup vs baseline: 4.9055x; 4.9055x over previous
import functools

import jax
import jax.numpy as jnp
from jax.experimental import pallas as pl
from jax.experimental.pallas import tpu as pltpu

_NUM_EXPERTS = 8


def _router_kernel(x_ref, wt_ref, gates_ref, idx_ref, loss_ref, acc_ref, *,
                   nblocks, ntokens):
    i = pl.program_id(0)
    logits = jnp.dot(x_ref[...], wt_ref[...],
                     preferred_element_type=jnp.float32)
    lt = logits.T
    bt = lt.shape[1]
    e = jax.lax.broadcasted_iota(jnp.int32, (_NUM_EXPERTS, bt), 0)

    m1 = jnp.max(lt, axis=0, keepdims=True)
    i1 = jnp.min(jnp.where(lt == m1, e, _NUM_EXPERTS), axis=0, keepdims=True)
    masked = jnp.where(e == i1, -jnp.inf, lt)
    m2 = jnp.max(masked, axis=0, keepdims=True)
    i2 = jnp.min(jnp.where(masked == m2, e, _NUM_EXPERTS), axis=0,
                 keepdims=True)

    ed = jnp.exp(m2 - m1)
    g2 = ed / (1.0 + ed)
    g1 = 1.0 - g2

    gt = jnp.where(e == i1, g1, jnp.where(e == i2, g2, jnp.float32(0.0)))
    gates_ref[...] = gt.T
    idx_ref[...] = jnp.concatenate([i1, i2], axis=0).T

    @pl.when(i == 0)
    def _init():
        acc_ref[...] = jnp.zeros_like(acc_ref)

    acc_ref[...] += jnp.sum(gt, axis=1, keepdims=True)

    @pl.when(i == nblocks - 1)
    def _finish():
        usage = acc_ref[...] / jnp.float32(ntokens)
        log_usage = jnp.maximum(jnp.log(usage), -1e9)
        u = jnp.float32(1.0 / _NUM_EXPERTS)
        loss_ref[...] = jnp.sum(u * (jnp.log(u) - log_usage)).reshape(1, 1)


def kernel(input_tensor, W):
    B, S, D = input_tensor.shape
    E = W.shape[0]
    n = B * S
    x = input_tensor.reshape(n, D)
    wt = W.T

    bt = 4096
    nblocks = n // bt

    gates, idx, loss = pl.pallas_call(
        functools.partial(_router_kernel, nblocks=nblocks, ntokens=n),
        grid=(nblocks,),
        in_specs=[
            pl.BlockSpec((bt, D), lambda i: (i, 0)),
            pl.BlockSpec((D, E), lambda i: (0, 0)),
        ],
        out_specs=[
            pl.BlockSpec((bt, E), lambda i: (i, 0)),
            pl.BlockSpec((bt, 2), lambda i: (i, 0)),
            pl.BlockSpec((1, 1), lambda i: (0, 0)),
        ],
        out_shape=[
            jax.ShapeDtypeStruct((n, E), jnp.float32),
            jax.ShapeDtypeStruct((n, 2), jnp.int32),
            jax.ShapeDtypeStruct((1, 1), jnp.float32),
        ],
        scratch_shapes=[pltpu.VMEM((E, 1), jnp.float32)],
    )(x, wt)

    return (gates.reshape(B, S, E), idx.reshape(B, S, 2), loss.reshape(()))
